# baseline (speedup 1.0000x reference)
import functools

import jax
import jax.numpy as jnp
from jax import lax
from jax.experimental import pallas as pl
from jax.experimental.pallas import tpu as pltpu
from jax.experimental.pallas import tpu_sc as plsc

T = 2048
H = 768
E = 8
I = 3072
K = 2
AUX_W = 0.01

B = 128
NT = (T * K) // B + E
NP = NT * B

NC = 2
NS = 16
NW = NC * NS
TW = T // NW


def _router_body(x_ref, gw_ref, w0_ref, w1_ref, aux_ref, pos0_ref, pos1_ref,
                 gid_ref, nrows_ref, first_ref, nxt_ref, slot_ref):
    x = x_ref[...]
    gw = gw_ref[...]
    logits = lax.dot_general(gw, x, (((1,), (1,)), ((), ())),
                             preferred_element_type=jnp.float32)
    mx = jnp.max(logits, axis=0, keepdims=True)
    ex = jnp.exp(logits - mx)
    probs = ex / jnp.sum(ex, axis=0, keepdims=True)

    eio = lax.broadcasted_iota(jnp.int32, (E, T), 0)
    p1 = jnp.max(probs, axis=0, keepdims=True)
    i1 = jnp.min(jnp.where(probs == p1, eio, E), axis=0, keepdims=True)
    a = (eio == i1).astype(jnp.float32)
    probs_m = jnp.where(a > 0.0, -jnp.inf, probs)
    p2 = jnp.max(probs_m, axis=0, keepdims=True)
    i2 = jnp.min(jnp.where(probs_m == p2, eio, E), axis=0, keepdims=True)
    b = (eio == i2).astype(jnp.float32)

    s12 = p1 + p2
    w0_ref[...] = (p1 / s12).reshape(T)
    w1_ref[...] = (p2 / s12).reshape(T)

    counts = jnp.sum(a + b, axis=1, keepdims=True)
    pm = jnp.sum(probs, axis=1, keepdims=True) * (1.0 / T)
    aux_ref[...] = jnp.sum((counts * (1.0 / (T * K))) * pm, axis=0,
                           keepdims=True) * (AUX_W * E)

    m = (a + b).astype(jnp.bfloat16)
    ri = lax.broadcasted_iota(jnp.int32, (T, T), 0)
    ci = lax.broadcasted_iota(jnp.int32, (T, T), 1)
    tri = (ri <= ci).astype(jnp.bfloat16)
    csum = lax.dot_general(m, tri, (((1,), (0,)), ((), ())),
                           preferred_element_type=jnp.float32)

    ci32 = counts.astype(jnp.int32)
    ntiles = (ci32 + (B - 1)) // B
    cumtiles = []
    acc = jnp.zeros((1, 1), jnp.int32)
    for e in range(E):
        cumtiles.append(acc)
        acc = acc + ntiles[e:e + 1, :]

    pos0 = jnp.zeros((1, T), jnp.float32)
    pos1 = jnp.zeros((1, T), jnp.float32)
    for e in range(E):
        off = (cumtiles[e] * B).astype(jnp.float32)
        ce = csum[e:e + 1, :]
        pos0 = pos0 + a[e:e + 1, :] * (off + ce - 1.0)
        pos1 = pos1 + b[e:e + 1, :] * (off + ce - 1.0)
    pos0_ref[...] = pos0.astype(jnp.int32).reshape(T)
    pos1_ref[...] = pos1.astype(jnp.int32).reshape(T)

    tio = lax.broadcasted_iota(jnp.int32, (1, NT), 1)
    gid = jnp.zeros((1, NT), jnp.int32)
    for e in range(1, E):
        gid = gid + (tio >= cumtiles[e]).astype(jnp.int32)
    lastg = jnp.zeros((1, 1), jnp.int32)
    for e in range(1, E):
        lastg = jnp.maximum(lastg, jnp.where(ci32[e:e + 1, :] > 0, e, 0))
    gid = jnp.minimum(gid, lastg)
    nrows = jnp.zeros((1, NT), jnp.int32)
    for e in range(E):
        r = ci32[e:e + 1, :] - (tio - cumtiles[e]) * B
        nrows = nrows + jnp.where(gid == e, jnp.clip(r, 0, B), 0)
    gid_ref[...] = gid.reshape(NT)
    nrows_ref[...] = nrows.reshape(NT)

    gprev = jnp.concatenate([jnp.full((1, 1), -1, jnp.int32), gid[:, :-1]],
                            axis=1)
    first = (gid != gprev).astype(jnp.int32)
    grp = first
    sh = 1
    while sh < NT:
        grp = grp + jnp.concatenate(
            [jnp.zeros((1, sh), jnp.int32), grp[:, :-sh]], axis=1)
        sh *= 2
    slot = (grp - 1) % 2
    nxte = [None] * E
    nxtv = [None] * E
    nxte[E - 1] = jnp.full((1, 1), E - 1, jnp.int32)
    nxtv[E - 1] = jnp.zeros((1, 1), jnp.int32)
    for e in range(E - 2, -1, -1):
        has = (ntiles[e + 1:e + 2, :] > 0)
        nxte[e] = jnp.where(has, e + 1, nxte[e + 1])
        nxtv[e] = jnp.where(has, 1, nxtv[e + 1])
    nxt = jnp.zeros((1, NT), jnp.int32)
    for e in range(E):
        nxt = nxt + jnp.where(gid == e,
                              jnp.where(nxtv[e] > 0, nxte[e], e), 0)
    first_ref[...] = first.reshape(NT)
    nxt_ref[...] = nxt.reshape(NT)
    slot_ref[...] = slot.reshape(NT)


def _router(x, gate_w):
    return pl.pallas_call(
        _router_body,
        out_shape=[
            jax.ShapeDtypeStruct((T,), jnp.float32),
            jax.ShapeDtypeStruct((T,), jnp.float32),
            jax.ShapeDtypeStruct((1, 1), jnp.float32),
            jax.ShapeDtypeStruct((T,), jnp.int32),
            jax.ShapeDtypeStruct((T,), jnp.int32),
            jax.ShapeDtypeStruct((NT,), jnp.int32),
            jax.ShapeDtypeStruct((NT,), jnp.int32),
            jax.ShapeDtypeStruct((NT,), jnp.int32),
            jax.ShapeDtypeStruct((NT,), jnp.int32),
            jax.ShapeDtypeStruct((NT,), jnp.int32),
        ],
    )(x, gate_w)


def _dispatch_body(x_hbm, pos0_hbm, pos1_hbm, xs_hbm,
                   idx0_v, idx1_v, rows_v, sem0, sem1):
    wid = lax.axis_index("s") * NC + lax.axis_index("c")
    base = wid * TW
    pltpu.sync_copy(pos0_hbm.at[pl.ds(base, TW)], idx0_v)
    pltpu.sync_copy(pos1_hbm.at[pl.ds(base, TW)], idx1_v)
    pltpu.sync_copy(x_hbm.at[pl.ds(base, TW)], rows_v)
    c0 = pltpu.async_copy(rows_v, xs_hbm.at[idx0_v], sem0)
    c1 = pltpu.async_copy(rows_v, xs_hbm.at[idx1_v], sem1)
    c0.wait()
    c1.wait()


def _dispatch(x, pos0, pos1):
    mesh = plsc.VectorSubcoreMesh(core_axis_name="c", subcore_axis_name="s")
    return pl.kernel(
        _dispatch_body,
        out_type=jax.ShapeDtypeStruct((NP, H), jnp.float32),
        mesh=mesh,
        scratch_types=[
            pltpu.VMEM((TW,), jnp.int32),
            pltpu.VMEM((TW,), jnp.int32),
            pltpu.VMEM((TW, H), jnp.float32),
            pltpu.SemaphoreType.DMA,
            pltpu.SemaphoreType.DMA,
        ],
    )(x, pos0, pos1)


def _gelu(h):
    inner = 0.7978845608028654 * (h + 0.044715 * (h * h * h))
    return 0.5 * h * (1.0 + jnp.tanh(inner))


I2 = I // 2


def _mlp_body(gid_ref, nrows_ref, first_ref, nxt_ref, slot_ref,
              xs_ref, w1_hbm, w2_hbm, os_ref, w1buf, w2buf, sem1, sem2):
    t = pl.program_id(0)
    sl = slot_ref[t]

    def cp1(e, sref):
        return pltpu.make_async_copy(w1_hbm.at[pl.ds(e, 1)],
                                     w1buf.at[pl.ds(sref, 1)], sem1.at[sref])

    def cp2(e, sref):
        return pltpu.make_async_copy(w2_hbm.at[pl.ds(e, 1)],
                                     w2buf.at[pl.ds(sref, 1)], sem2.at[sref])

    @pl.when(t == 0)
    def _():
        cp1(gid_ref[0], 0).start()
        cp2(gid_ref[0], 0).start()

    @pl.when((t == 0) & (nxt_ref[0] != gid_ref[0]))
    def _():
        cp1(nxt_ref[0], 1).start()
        cp2(nxt_ref[0], 1).start()

    @pl.when((first_ref[t] == 1) & (t > 0) & (nxt_ref[t] != gid_ref[t]))
    def _():
        cp1(nxt_ref[t], 1 - sl).start()
        cp2(nxt_ref[t], 1 - sl).start()

    @pl.when(first_ref[t] == 1)
    def _():
        cp1(gid_ref[t], sl).wait()
        cp2(gid_ref[t], sl).wait()

    @pl.when(nrows_ref[t] > 0)
    def _():
        xb = xs_ref[...].astype(jnp.bfloat16)
        dn = (((1,), (0,)), ((), ()))
        acc = jnp.zeros((B, H), jnp.float32)
        for half in range(2):
            w1h = w1buf[sl, :, pl.ds(half * I2, I2)].astype(jnp.bfloat16)
            h = lax.dot_general(xb, w1h, dn,
                                preferred_element_type=jnp.float32)
            g = _gelu(h).astype(jnp.bfloat16)
            w2h = w2buf[sl, pl.ds(half * I2, I2), :].astype(jnp.bfloat16)
            acc = acc + lax.dot_general(g, w2h, dn,
                                        preferred_element_type=jnp.float32)
        os_ref[...] = acc


def _groupmlp(gid, nrows, first, nxt, slot, xs, w1, w2):
    grid_spec = pltpu.PrefetchScalarGridSpec(
        num_scalar_prefetch=5,
        grid=(NT,),
        in_specs=[
            pl.BlockSpec((B, H), lambda t, *_: (t, 0)),
            pl.BlockSpec(memory_space=pl.ANY),
            pl.BlockSpec(memory_space=pl.ANY),
        ],
        out_specs=pl.BlockSpec((B, H), lambda t, *_: (t, 0)),
        scratch_shapes=[
            pltpu.VMEM((2, H, I), jnp.float32),
            pltpu.VMEM((2, I, H), jnp.float32),
            pltpu.SemaphoreType.DMA((2,)),
            pltpu.SemaphoreType.DMA((2,)),
        ],
    )
    return pl.pallas_call(
        _mlp_body,
        grid_spec=grid_spec,
        out_shape=jax.ShapeDtypeStruct((NP, H), jnp.float32),
    )(gid, nrows, first, nxt, slot, xs, w1, w2)


def _combine_body(os_hbm, pos0_hbm, pos1_hbm, w0_hbm, w1_hbm, out_hbm,
                  idx0_v, idx1_v, wb0_v, wb1_v, rows0_v, rows1_v, sem0, sem1):
    wid = lax.axis_index("s") * NC + lax.axis_index("c")
    base = wid * TW
    pltpu.sync_copy(pos0_hbm.at[pl.ds(base, TW)], idx0_v)
    pltpu.sync_copy(pos1_hbm.at[pl.ds(base, TW)], idx1_v)
    pltpu.sync_copy(w0_hbm.at[pl.ds(base, TW)], wb0_v)
    pltpu.sync_copy(w1_hbm.at[pl.ds(base, TW)], wb1_v)
    c0 = pltpu.async_copy(os_hbm.at[idx0_v], rows0_v, sem0)
    c1 = pltpu.async_copy(os_hbm.at[idx1_v], rows1_v, sem1)
    c0.wait()
    c1.wait()

    def row(r, carry):
        chunk = (r // 16) * 16
        off16 = jnp.zeros((16,), jnp.int32) + (r - chunk)
        wv0 = wb0_v[pl.ds(chunk, 16)].at[off16].get(mode="promise_in_bounds")
        wv1 = wb1_v[pl.ds(chunk, 16)].at[off16].get(mode="promise_in_bounds")
        for c in range(H // 16):
            sl = pl.ds(c * 16, 16)
            rows0_v[r, sl] = rows0_v[r, sl] * wv0 + rows1_v[r, sl] * wv1
        return carry

    lax.fori_loop(0, TW, row, 0)
    pltpu.sync_copy(rows0_v, out_hbm.at[pl.ds(base, TW)])


def _combine(os_, pos0, pos1, w0r, w1r):
    mesh = plsc.VectorSubcoreMesh(core_axis_name="c", subcore_axis_name="s")
    return pl.kernel(
        _combine_body,
        out_type=jax.ShapeDtypeStruct((T, H), jnp.float32),
        mesh=mesh,
        scratch_types=[
            pltpu.VMEM((TW,), jnp.int32),
            pltpu.VMEM((TW,), jnp.int32),
            pltpu.VMEM((TW,), jnp.float32),
            pltpu.VMEM((TW,), jnp.float32),
            pltpu.VMEM((TW, H), jnp.float32),
            pltpu.VMEM((TW, H), jnp.float32),
            pltpu.SemaphoreType.DMA,
            pltpu.SemaphoreType.DMA,
        ],
    )(os_, pos0, pos1, w0r, w1r)


def kernel(x, gate_w, w1, w2):
    (w0r, w1r, aux, pos0, pos1, gid, nrows, first, nxt,
     slot) = _router(x, gate_w)
    xs = _dispatch(x, pos0, pos1)
    os_ = _groupmlp(gid, nrows, first, nxt, slot, xs, w1, w2)
    out = _combine(os_, pos0, pos1, w0r, w1r)
    return out, aux[0, 0]


# speedup vs baseline: 1.0839x; 1.0839x over previous
import functools

import jax
import jax.numpy as jnp
from jax import lax
from jax.experimental import pallas as pl
from jax.experimental.pallas import tpu as pltpu
from jax.experimental.pallas import tpu_sc as plsc

T = 2048
H = 768
E = 8
I = 3072
K = 2
AUX_W = 0.01

B = 256
NT = (T * K) // B + E
NP = NT * B

NC = 2
NS = 16
NW = NC * NS
TW = T // NW


def _router_body(x_ref, gw_ref, w0_ref, w1_ref, aux_ref, pos0_ref, pos1_ref,
                 gid_ref, nrows_ref, first_ref, nxt_ref, slot_ref):
    x = x_ref[...]
    gw = gw_ref[...]
    logits = lax.dot_general(gw, x, (((1,), (1,)), ((), ())),
                             preferred_element_type=jnp.float32)
    mx = jnp.max(logits, axis=0, keepdims=True)
    ex = jnp.exp(logits - mx)
    probs = ex / jnp.sum(ex, axis=0, keepdims=True)

    eio = lax.broadcasted_iota(jnp.int32, (E, T), 0)
    p1 = jnp.max(probs, axis=0, keepdims=True)
    i1 = jnp.min(jnp.where(probs == p1, eio, E), axis=0, keepdims=True)
    a = (eio == i1).astype(jnp.float32)
    probs_m = jnp.where(a > 0.0, -jnp.inf, probs)
    p2 = jnp.max(probs_m, axis=0, keepdims=True)
    i2 = jnp.min(jnp.where(probs_m == p2, eio, E), axis=0, keepdims=True)
    b = (eio == i2).astype(jnp.float32)

    s12 = p1 + p2
    w0_ref[...] = (p1 / s12).reshape(T)
    w1_ref[...] = (p2 / s12).reshape(T)

    counts = jnp.sum(a + b, axis=1, keepdims=True)
    pm = jnp.sum(probs, axis=1, keepdims=True) * (1.0 / T)
    aux_ref[...] = jnp.sum((counts * (1.0 / (T * K))) * pm, axis=0,
                           keepdims=True) * (AUX_W * E)

    m = (a + b).astype(jnp.bfloat16)
    ri = lax.broadcasted_iota(jnp.int32, (T, T), 0)
    ci = lax.broadcasted_iota(jnp.int32, (T, T), 1)
    tri = (ri <= ci).astype(jnp.bfloat16)
    csum = lax.dot_general(m, tri, (((1,), (0,)), ((), ())),
                           preferred_element_type=jnp.float32)

    ci32 = counts.astype(jnp.int32)
    ntiles = (ci32 + (B - 1)) // B
    cumtiles = []
    acc = jnp.zeros((1, 1), jnp.int32)
    for e in range(E):
        cumtiles.append(acc)
        acc = acc + ntiles[e:e + 1, :]

    pos0 = jnp.zeros((1, T), jnp.float32)
    pos1 = jnp.zeros((1, T), jnp.float32)
    for e in range(E):
        off = (cumtiles[e] * B).astype(jnp.float32)
        ce = csum[e:e + 1, :]
        pos0 = pos0 + a[e:e + 1, :] * (off + ce - 1.0)
        pos1 = pos1 + b[e:e + 1, :] * (off + ce - 1.0)
    pos0_ref[...] = pos0.astype(jnp.int32).reshape(T)
    pos1_ref[...] = pos1.astype(jnp.int32).reshape(T)

    tio = lax.broadcasted_iota(jnp.int32, (1, NT), 1)
    gid = jnp.zeros((1, NT), jnp.int32)
    for e in range(1, E):
        gid = gid + (tio >= cumtiles[e]).astype(jnp.int32)
    lastg = jnp.zeros((1, 1), jnp.int32)
    for e in range(1, E):
        lastg = jnp.maximum(lastg, jnp.where(ci32[e:e + 1, :] > 0, e, 0))
    gid = jnp.minimum(gid, lastg)
    nrows = jnp.zeros((1, NT), jnp.int32)
    for e in range(E):
        r = ci32[e:e + 1, :] - (tio - cumtiles[e]) * B
        nrows = nrows + jnp.where(gid == e, jnp.clip(r, 0, B), 0)
    gid_ref[...] = gid.reshape(NT)
    nrows_ref[...] = nrows.reshape(NT)

    gprev = jnp.concatenate([jnp.full((1, 1), -1, jnp.int32), gid[:, :-1]],
                            axis=1)
    first = (gid != gprev).astype(jnp.int32)
    grp = first
    sh = 1
    while sh < NT:
        grp = grp + jnp.concatenate(
            [jnp.zeros((1, sh), jnp.int32), grp[:, :-sh]], axis=1)
        sh *= 2
    slot = (grp - 1) % 2
    nxte = [None] * E
    nxtv = [None] * E
    nxte[E - 1] = jnp.full((1, 1), E - 1, jnp.int32)
    nxtv[E - 1] = jnp.zeros((1, 1), jnp.int32)
    for e in range(E - 2, -1, -1):
        has = (ntiles[e + 1:e + 2, :] > 0)
        nxte[e] = jnp.where(has, e + 1, nxte[e + 1])
        nxtv[e] = jnp.where(has, 1, nxtv[e + 1])
    nxt = jnp.zeros((1, NT), jnp.int32)
    for e in range(E):
        nxt = nxt + jnp.where(gid == e,
                              jnp.where(nxtv[e] > 0, nxte[e], e), 0)
    first_ref[...] = first.reshape(NT)
    nxt_ref[...] = nxt.reshape(NT)
    slot_ref[...] = slot.reshape(NT)


def _router(x, gate_w):
    return pl.pallas_call(
        _router_body,
        out_shape=[
            jax.ShapeDtypeStruct((T,), jnp.float32),
            jax.ShapeDtypeStruct((T,), jnp.float32),
            jax.ShapeDtypeStruct((1, 1), jnp.float32),
            jax.ShapeDtypeStruct((T,), jnp.int32),
            jax.ShapeDtypeStruct((T,), jnp.int32),
            jax.ShapeDtypeStruct((NT,), jnp.int32),
            jax.ShapeDtypeStruct((NT,), jnp.int32),
            jax.ShapeDtypeStruct((NT,), jnp.int32),
            jax.ShapeDtypeStruct((NT,), jnp.int32),
            jax.ShapeDtypeStruct((NT,), jnp.int32),
        ],
    )(x, gate_w)


def _dispatch_body(x_hbm, pos0_hbm, pos1_hbm, xs_hbm,
                   idx0_v, idx1_v, rows_v, sem0, sem1, semi):
    wid = lax.axis_index("s") * NC + lax.axis_index("c")
    base = wid * TW
    i0 = pltpu.async_copy(pos0_hbm.at[pl.ds(base, TW)], idx0_v, semi)
    i1 = pltpu.async_copy(pos1_hbm.at[pl.ds(base, TW)], idx1_v, semi)
    ix = pltpu.async_copy(x_hbm.at[pl.ds(base, TW)], rows_v, semi)
    i0.wait()
    i1.wait()
    ix.wait()
    c0 = pltpu.async_copy(rows_v, xs_hbm.at[idx0_v], sem0)
    c1 = pltpu.async_copy(rows_v, xs_hbm.at[idx1_v], sem1)
    c0.wait()
    c1.wait()


def _dispatch(x, pos0, pos1):
    mesh = plsc.VectorSubcoreMesh(core_axis_name="c", subcore_axis_name="s")
    return pl.kernel(
        _dispatch_body,
        out_type=jax.ShapeDtypeStruct((NP, H), jnp.float32),
        mesh=mesh,
        scratch_types=[
            pltpu.VMEM((TW,), jnp.int32),
            pltpu.VMEM((TW,), jnp.int32),
            pltpu.VMEM((TW, H), jnp.float32),
            pltpu.SemaphoreType.DMA,
            pltpu.SemaphoreType.DMA,
            pltpu.SemaphoreType.DMA,
        ],
    )(x, pos0, pos1)


def _gelu(h):
    inner = 0.7978845608028654 * (h + 0.044715 * (h * h * h))
    return 0.5 * h * (1.0 + jnp.tanh(inner))


I2 = I // 2


def _mlp_body(gid_ref, nrows_ref, first_ref, nxt_ref, slot_ref,
              xs_ref, w1_hbm, w2_hbm, os_ref, w1buf, w2buf, sem1, sem2):
    t = pl.program_id(0)
    sl = slot_ref[t]

    def cp1(e, sref, half):
        return pltpu.make_async_copy(
            w1_hbm.at[pl.ds(e, 1), :, pl.ds(half * I2, I2)],
            w1buf.at[pl.ds(sref, 1), :, pl.ds(half * I2, I2)],
            sem1.at[sref, half])

    def cp2(e, sref, half):
        return pltpu.make_async_copy(
            w2_hbm.at[pl.ds(e, 1), pl.ds(half * I2, I2), :],
            w2buf.at[pl.ds(sref, 1), pl.ds(half * I2, I2), :],
            sem2.at[sref, half])

    def start_group(e, sref):
        cp1(e, sref, 0).start()
        cp2(e, sref, 0).start()
        cp1(e, sref, 1).start()
        cp2(e, sref, 1).start()

    @pl.when(t == 0)
    def _():
        start_group(gid_ref[0], 0)

    @pl.when((t == 0) & (nxt_ref[0] != gid_ref[0]))
    def _():
        start_group(nxt_ref[0], 1)

    @pl.when((first_ref[t] == 1) & (t > 0) & (nxt_ref[t] != gid_ref[t]))
    def _():
        start_group(nxt_ref[t], 1 - sl)

    @pl.when(nrows_ref[t] > 0)
    def _():
        xb = xs_ref[...].astype(jnp.bfloat16)
        dn = (((1,), (0,)), ((), ()))
        acc = jnp.zeros((B, H), jnp.float32)
        for half in range(2):
            @pl.when(first_ref[t] == 1)
            def _():
                cp1(gid_ref[t], sl, half).wait()
                cp2(gid_ref[t], sl, half).wait()
            w1h = w1buf[sl, :, pl.ds(half * I2, I2)].astype(jnp.bfloat16)
            h = lax.dot_general(xb, w1h, dn,
                                preferred_element_type=jnp.float32)
            g = _gelu(h).astype(jnp.bfloat16)
            w2h = w2buf[sl, pl.ds(half * I2, I2), :].astype(jnp.bfloat16)
            acc = acc + lax.dot_general(g, w2h, dn,
                                        preferred_element_type=jnp.float32)
        os_ref[...] = acc


def _groupmlp(gid, nrows, first, nxt, slot, xs, w1, w2):
    grid_spec = pltpu.PrefetchScalarGridSpec(
        num_scalar_prefetch=5,
        grid=(NT,),
        in_specs=[
            pl.BlockSpec((B, H), lambda t, *_: (t, 0)),
            pl.BlockSpec(memory_space=pl.ANY),
            pl.BlockSpec(memory_space=pl.ANY),
        ],
        out_specs=pl.BlockSpec((B, H), lambda t, *_: (t, 0)),
        scratch_shapes=[
            pltpu.VMEM((2, H, I), jnp.float32),
            pltpu.VMEM((2, I, H), jnp.float32),
            pltpu.SemaphoreType.DMA((2, 2)),
            pltpu.SemaphoreType.DMA((2, 2)),
        ],
    )
    return pl.pallas_call(
        _mlp_body,
        grid_spec=grid_spec,
        out_shape=jax.ShapeDtypeStruct((NP, H), jnp.float32),
    )(gid, nrows, first, nxt, slot, xs, w1, w2)


def _combine_body(os_hbm, pos0_hbm, pos1_hbm, w0_hbm, w1_hbm, out_hbm,
                  idx0_v, idx1_v, wb0_v, wb1_v, rows0_v, rows1_v, sem0, sem1):
    wid = lax.axis_index("s") * NC + lax.axis_index("c")
    base = wid * TW
    pltpu.sync_copy(pos0_hbm.at[pl.ds(base, TW)], idx0_v)
    pltpu.sync_copy(pos1_hbm.at[pl.ds(base, TW)], idx1_v)
    pltpu.sync_copy(w0_hbm.at[pl.ds(base, TW)], wb0_v)
    pltpu.sync_copy(w1_hbm.at[pl.ds(base, TW)], wb1_v)
    c0 = pltpu.async_copy(os_hbm.at[idx0_v], rows0_v, sem0)
    c1 = pltpu.async_copy(os_hbm.at[idx1_v], rows1_v, sem1)
    c0.wait()
    c1.wait()

    def row(r, carry):
        chunk = (r // 16) * 16
        off16 = jnp.zeros((16,), jnp.int32) + (r - chunk)
        wv0 = wb0_v[pl.ds(chunk, 16)].at[off16].get(mode="promise_in_bounds")
        wv1 = wb1_v[pl.ds(chunk, 16)].at[off16].get(mode="promise_in_bounds")
        for c in range(H // 16):
            sl = pl.ds(c * 16, 16)
            rows0_v[r, sl] = rows0_v[r, sl] * wv0 + rows1_v[r, sl] * wv1
        return carry

    lax.fori_loop(0, TW, row, 0)
    pltpu.sync_copy(rows0_v, out_hbm.at[pl.ds(base, TW)])


def _combine(os_, pos0, pos1, w0r, w1r):
    mesh = plsc.VectorSubcoreMesh(core_axis_name="c", subcore_axis_name="s")
    return pl.kernel(
        _combine_body,
        out_type=jax.ShapeDtypeStruct((T, H), jnp.float32),
        mesh=mesh,
        scratch_types=[
            pltpu.VMEM((TW,), jnp.int32),
            pltpu.VMEM((TW,), jnp.int32),
            pltpu.VMEM((TW,), jnp.float32),
            pltpu.VMEM((TW,), jnp.float32),
            pltpu.VMEM((TW, H), jnp.float32),
            pltpu.VMEM((TW, H), jnp.float32),
            pltpu.SemaphoreType.DMA,
            pltpu.SemaphoreType.DMA,
        ],
    )(os_, pos0, pos1, w0r, w1r)


def kernel(x, gate_w, w1, w2):
    (w0r, w1r, aux, pos0, pos1, gid, nrows, first, nxt,
     slot) = _router(x, gate_w)
    xs = _dispatch(x, pos0, pos1)
    os_ = _groupmlp(gid, nrows, first, nxt, slot, xs, w1, w2)
    out = _combine(os_, pos0, pos1, w0r, w1r)
    return out, aux[0, 0]


# speedup vs baseline: 1.1896x; 1.0975x over previous
import functools

import jax
import jax.numpy as jnp
from jax import lax
from jax.experimental import pallas as pl
from jax.experimental.pallas import tpu as pltpu
from jax.experimental.pallas import tpu_sc as plsc

T = 2048
H = 768
E = 8
I = 3072
K = 2
AUX_W = 0.01

B = 256
NT = (T * K) // B + E
NP = NT * B

NC = 2
NS = 16
NW = NC * NS
TW = T // NW


def _router_body(x_ref, gw_ref, w0_ref, w1_ref, aux_ref, pos0_ref, pos1_ref,
                 gid_ref, nrows_ref, first_ref, nxt_ref, slot_ref):
    x = x_ref[...]
    gw = gw_ref[...]
    logits = lax.dot_general(gw, x, (((1,), (1,)), ((), ())),
                             preferred_element_type=jnp.float32)
    mx = jnp.max(logits, axis=0, keepdims=True)
    ex = jnp.exp(logits - mx)
    probs = ex / jnp.sum(ex, axis=0, keepdims=True)

    eio = lax.broadcasted_iota(jnp.int32, (E, T), 0)
    p1 = jnp.max(probs, axis=0, keepdims=True)
    i1 = jnp.min(jnp.where(probs == p1, eio, E), axis=0, keepdims=True)
    a = (eio == i1).astype(jnp.float32)
    probs_m = jnp.where(a > 0.0, -jnp.inf, probs)
    p2 = jnp.max(probs_m, axis=0, keepdims=True)
    i2 = jnp.min(jnp.where(probs_m == p2, eio, E), axis=0, keepdims=True)
    b = (eio == i2).astype(jnp.float32)

    s12 = p1 + p2
    w0_ref[...] = (p1 / s12).reshape(T)
    w1_ref[...] = (p2 / s12).reshape(T)

    counts = jnp.sum(a + b, axis=1, keepdims=True)
    pm = jnp.sum(probs, axis=1, keepdims=True) * (1.0 / T)
    aux_ref[...] = jnp.sum((counts * (1.0 / (T * K))) * pm, axis=0,
                           keepdims=True) * (AUX_W * E)

    m = (a + b).astype(jnp.bfloat16)
    ri = lax.broadcasted_iota(jnp.int32, (T, T), 0)
    ci = lax.broadcasted_iota(jnp.int32, (T, T), 1)
    tri = (ri <= ci).astype(jnp.bfloat16)
    csum = lax.dot_general(m, tri, (((1,), (0,)), ((), ())),
                           preferred_element_type=jnp.float32)

    ci32 = counts.astype(jnp.int32)
    ntiles = (ci32 + (B - 1)) // B
    cumtiles = []
    acc = jnp.zeros((1, 1), jnp.int32)
    for e in range(E):
        cumtiles.append(acc)
        acc = acc + ntiles[e:e + 1, :]

    pos0 = jnp.zeros((1, T), jnp.float32)
    pos1 = jnp.zeros((1, T), jnp.float32)
    for e in range(E):
        off = (cumtiles[e] * B).astype(jnp.float32)
        ce = csum[e:e + 1, :]
        pos0 = pos0 + a[e:e + 1, :] * (off + ce - 1.0)
        pos1 = pos1 + b[e:e + 1, :] * (off + ce - 1.0)
    pos0_ref[...] = pos0.astype(jnp.int32).reshape(T)
    pos1_ref[...] = pos1.astype(jnp.int32).reshape(T)

    tio = lax.broadcasted_iota(jnp.int32, (1, NT), 1)
    gid = jnp.zeros((1, NT), jnp.int32)
    for e in range(1, E):
        gid = gid + (tio >= cumtiles[e]).astype(jnp.int32)
    lastg = jnp.zeros((1, 1), jnp.int32)
    for e in range(1, E):
        lastg = jnp.maximum(lastg, jnp.where(ci32[e:e + 1, :] > 0, e, 0))
    gid = jnp.minimum(gid, lastg)
    nrows = jnp.zeros((1, NT), jnp.int32)
    for e in range(E):
        r = ci32[e:e + 1, :] - (tio - cumtiles[e]) * B
        nrows = nrows + jnp.where(gid == e, jnp.clip(r, 0, B), 0)
    gid_ref[...] = gid.reshape(NT)
    nrows_ref[...] = nrows.reshape(NT)

    gprev = jnp.concatenate([jnp.full((1, 1), -1, jnp.int32), gid[:, :-1]],
                            axis=1)
    first = (gid != gprev).astype(jnp.int32)
    grp = first
    sh = 1
    while sh < NT:
        grp = grp + jnp.concatenate(
            [jnp.zeros((1, sh), jnp.int32), grp[:, :-sh]], axis=1)
        sh *= 2
    slot = (grp - 1) % 2
    nxte = [None] * E
    nxtv = [None] * E
    nxte[E - 1] = jnp.full((1, 1), E - 1, jnp.int32)
    nxtv[E - 1] = jnp.zeros((1, 1), jnp.int32)
    for e in range(E - 2, -1, -1):
        has = (ntiles[e + 1:e + 2, :] > 0)
        nxte[e] = jnp.where(has, e + 1, nxte[e + 1])
        nxtv[e] = jnp.where(has, 1, nxtv[e + 1])
    nxt = jnp.zeros((1, NT), jnp.int32)
    for e in range(E):
        nxt = nxt + jnp.where(gid == e,
                              jnp.where(nxtv[e] > 0, nxte[e], e), 0)
    first_ref[...] = first.reshape(NT)
    nxt_ref[...] = nxt.reshape(NT)
    slot_ref[...] = slot.reshape(NT)


def _router(x, gate_w):
    return pl.pallas_call(
        _router_body,
        out_shape=[
            jax.ShapeDtypeStruct((T,), jnp.float32),
            jax.ShapeDtypeStruct((T,), jnp.float32),
            jax.ShapeDtypeStruct((1, 1), jnp.float32),
            jax.ShapeDtypeStruct((T,), jnp.int32),
            jax.ShapeDtypeStruct((T,), jnp.int32),
            jax.ShapeDtypeStruct((NT,), jnp.int32),
            jax.ShapeDtypeStruct((NT,), jnp.int32),
            jax.ShapeDtypeStruct((NT,), jnp.int32),
            jax.ShapeDtypeStruct((NT,), jnp.int32),
            jax.ShapeDtypeStruct((NT,), jnp.int32),
        ],
    )(x, gate_w)


def _dispatch_body(x_hbm, pos0_hbm, pos1_hbm, xs_hbm,
                   idx0_v, idx1_v, rows_v, sem0, sem1, semi):
    wid = lax.axis_index("s") * NC + lax.axis_index("c")
    base = wid * TW
    i0 = pltpu.async_copy(pos0_hbm.at[pl.ds(base, TW)], idx0_v, semi)
    i1 = pltpu.async_copy(pos1_hbm.at[pl.ds(base, TW)], idx1_v, semi)
    ix = pltpu.async_copy(x_hbm.at[pl.ds(base, TW)], rows_v, semi)
    i0.wait()
    i1.wait()
    ix.wait()
    c0 = pltpu.async_copy(rows_v, xs_hbm.at[idx0_v], sem0)
    c1 = pltpu.async_copy(rows_v, xs_hbm.at[idx1_v], sem1)
    c0.wait()
    c1.wait()


def _dispatch(x, pos0, pos1):
    mesh = plsc.VectorSubcoreMesh(core_axis_name="c", subcore_axis_name="s")
    return pl.kernel(
        _dispatch_body,
        out_type=jax.ShapeDtypeStruct((NP, H), jnp.float32),
        mesh=mesh,
        scratch_types=[
            pltpu.VMEM((TW,), jnp.int32),
            pltpu.VMEM((TW,), jnp.int32),
            pltpu.VMEM((TW, H), jnp.float32),
            pltpu.SemaphoreType.DMA,
            pltpu.SemaphoreType.DMA,
            pltpu.SemaphoreType.DMA,
        ],
    )(x, pos0, pos1)


def _gelu(h):
    inner = 0.7978845608028654 * (h + 0.044715 * (h * h * h))
    return 0.5 * h * (1.0 + jnp.tanh(inner))


I2 = I // 2


def _mlp_body(gid_ref, nrows_ref, first_ref, nxt_ref, slot_ref,
              xs_ref, w1_hbm, w2_hbm, os_ref, w1buf, w2buf, sem1, sem2):
    t = pl.program_id(0)
    sl = slot_ref[t]

    def cp1(e, sref):
        return pltpu.make_async_copy(w1_hbm.at[pl.ds(e, 1)],
                                     w1buf.at[pl.ds(sref, 1)], sem1.at[sref])

    def cp2(e, sref):
        return pltpu.make_async_copy(w2_hbm.at[pl.ds(e, 1)],
                                     w2buf.at[pl.ds(sref, 1)], sem2.at[sref])

    @pl.when(t == 0)
    def _():
        cp1(gid_ref[0], 0).start()
        cp2(gid_ref[0], 0).start()

    @pl.when((t == 0) & (nxt_ref[0] != gid_ref[0]))
    def _():
        cp1(nxt_ref[0], 1).start()
        cp2(nxt_ref[0], 1).start()

    @pl.when((first_ref[t] == 1) & (t > 0) & (nxt_ref[t] != gid_ref[t]))
    def _():
        cp1(nxt_ref[t], 1 - sl).start()
        cp2(nxt_ref[t], 1 - sl).start()

    @pl.when(first_ref[t] == 1)
    def _():
        cp1(gid_ref[t], sl).wait()
        cp2(gid_ref[t], sl).wait()

    @pl.when(nrows_ref[t] > 0)
    def _():
        xb = xs_ref[...].astype(jnp.bfloat16)
        dn = (((1,), (0,)), ((), ()))
        acc = jnp.zeros((B, H), jnp.float32)
        for half in range(2):
            w1h = w1buf[sl, :, pl.ds(half * I2, I2)].astype(jnp.bfloat16)
            h = lax.dot_general(xb, w1h, dn,
                                preferred_element_type=jnp.float32)
            g = _gelu(h).astype(jnp.bfloat16)
            w2h = w2buf[sl, pl.ds(half * I2, I2), :].astype(jnp.bfloat16)
            acc = acc + lax.dot_general(g, w2h, dn,
                                        preferred_element_type=jnp.float32)
        os_ref[...] = acc


def _groupmlp(gid, nrows, first, nxt, slot, xs, w1, w2):
    grid_spec = pltpu.PrefetchScalarGridSpec(
        num_scalar_prefetch=5,
        grid=(NT,),
        in_specs=[
            pl.BlockSpec((B, H), lambda t, *_: (t, 0)),
            pl.BlockSpec(memory_space=pl.ANY),
            pl.BlockSpec(memory_space=pl.ANY),
        ],
        out_specs=pl.BlockSpec((B, H), lambda t, *_: (t, 0)),
        scratch_shapes=[
            pltpu.VMEM((2, H, I), jnp.float32),
            pltpu.VMEM((2, I, H), jnp.float32),
            pltpu.SemaphoreType.DMA((2,)),
            pltpu.SemaphoreType.DMA((2,)),
        ],
    )
    return pl.pallas_call(
        _mlp_body,
        grid_spec=grid_spec,
        out_shape=jax.ShapeDtypeStruct((NP, H), jnp.float32),
    )(gid, nrows, first, nxt, slot, xs, w1, w2)


def _combine_body(os_hbm, pos0_hbm, pos1_hbm, w0_hbm, w1_hbm, out_hbm,
                  idx0_v, idx1_v, wb0_v, wb1_v, rows0_v, rows1_v, sem0, sem1):
    wid = lax.axis_index("s") * NC + lax.axis_index("c")
    base = wid * TW
    pltpu.sync_copy(pos0_hbm.at[pl.ds(base, TW)], idx0_v)
    pltpu.sync_copy(pos1_hbm.at[pl.ds(base, TW)], idx1_v)
    pltpu.sync_copy(w0_hbm.at[pl.ds(base, TW)], wb0_v)
    pltpu.sync_copy(w1_hbm.at[pl.ds(base, TW)], wb1_v)
    c0 = pltpu.async_copy(os_hbm.at[idx0_v], rows0_v, sem0)
    c1 = pltpu.async_copy(os_hbm.at[idx1_v], rows1_v, sem1)
    c0.wait()
    c1.wait()

    def row(r, carry):
        chunk = (r // 16) * 16
        off16 = jnp.zeros((16,), jnp.int32) + (r - chunk)
        wv0 = wb0_v[pl.ds(chunk, 16)].at[off16].get(mode="promise_in_bounds")
        wv1 = wb1_v[pl.ds(chunk, 16)].at[off16].get(mode="promise_in_bounds")
        for c in range(H // 16):
            sl = pl.ds(c * 16, 16)
            rows0_v[r, sl] = rows0_v[r, sl] * wv0 + rows1_v[r, sl] * wv1
        return carry

    lax.fori_loop(0, TW, row, 0)
    pltpu.sync_copy(rows0_v, out_hbm.at[pl.ds(base, TW)])


def _combine(os_, pos0, pos1, w0r, w1r):
    mesh = plsc.VectorSubcoreMesh(core_axis_name="c", subcore_axis_name="s")
    return pl.kernel(
        _combine_body,
        out_type=jax.ShapeDtypeStruct((T, H), jnp.float32),
        mesh=mesh,
        scratch_types=[
            pltpu.VMEM((TW,), jnp.int32),
            pltpu.VMEM((TW,), jnp.int32),
            pltpu.VMEM((TW,), jnp.float32),
            pltpu.VMEM((TW,), jnp.float32),
            pltpu.VMEM((TW, H), jnp.float32),
            pltpu.VMEM((TW, H), jnp.float32),
            pltpu.SemaphoreType.DMA,
            pltpu.SemaphoreType.DMA,
        ],
    )(os_, pos0, pos1, w0r, w1r)


def kernel(x, gate_w, w1, w2):
    (w0r, w1r, aux, pos0, pos1, gid, nrows, first, nxt,
     slot) = _router(x, gate_w)
    xs = _dispatch(x, pos0, pos1)
    os_ = _groupmlp(gid, nrows, first, nxt, slot, xs, w1, w2)
    out = _combine(os_, pos0, pos1, w0r, w1r)
    return out, aux[0, 0]
